# baseline (speedup 1.0000x reference)
import functools

import jax
import jax.numpy as jnp
from jax import lax
from jax.experimental import pallas as pl

AUG_P = 0.5
DROP_P = 0.15
MIN_FEATURES = 1


def _mask_kernel(sT_ref, mT_ref, aug_ref, keepT_ref, *, F, B):
    m = mT_ref[...] > 0
    bits = lax.bitcast_convert_type(sT_ref[...], jnp.int32)
    bits = jnp.where(m, bits, jnp.int32(0x7FFFFFFF))

    n_avail = jnp.sum(m.astype(jnp.int32), axis=0, keepdims=True)
    k = (n_avail.astype(jnp.float32) * DROP_P).astype(jnp.int32)
    k = jnp.minimum(k, n_avail - MIN_FEATURES)
    aug = aug_ref[...] < AUG_P
    k = jnp.where((n_avail > MIN_FEATURES) & aug & (k > 0), k, 0)

    ans = jnp.zeros((1, B), jnp.int32)
    for bit in range(30, -1, -1):
        test = ans + jnp.int32(1 << bit)
        cnt = jnp.sum((bits < test).astype(jnp.int32), axis=0, keepdims=True)
        ans = jnp.where(cnt < k, test, ans)

    c_lt = jnp.sum((bits < ans).astype(jnp.int32), axis=0, keepdims=True)
    eq = bits == ans
    fi = lax.broadcasted_iota(jnp.int32, (F, F), 0)
    fj = lax.broadcasted_iota(jnp.int32, (F, F), 1)
    tril = (fj < fi).astype(jnp.float32)
    eq_before = jax.lax.dot(
        tril, eq.astype(jnp.float32), precision=jax.lax.Precision.HIGHEST
    ).astype(jnp.int32)
    drop = m & ((bits < ans) | (eq & ((c_lt + eq_before) < k)))
    keepT_ref[...] = 1.0 - drop.astype(jnp.float32)


def _copy_kernel(x_ref, k_ref, o_ref):
    o_ref[...] = x_ref[...] * k_ref[...]


def kernel(input_features, attention_mask):
    B, F, C = input_features.shape
    key = jax.random.key(42)
    k1, k2 = jax.random.split(key)
    aug_u = jax.random.uniform(k1, (B,)).reshape(1, B)
    scores = jax.random.uniform(k2, (B, F))

    sT = scores.T
    mT = attention_mask.astype(jnp.int32).T

    keepT = pl.pallas_call(
        functools.partial(_mask_kernel, F=F, B=B),
        out_shape=jax.ShapeDtypeStruct((F, B), jnp.float32),
    )(sT, mT, aug_u)

    keep_col = keepT.T.reshape(B * F, 1)
    x2 = input_features.reshape(B * F, C)

    ROWS = 2048
    grid = (B * F // ROWS,)
    out = pl.pallas_call(
        _copy_kernel,
        grid=grid,
        in_specs=[
            pl.BlockSpec((ROWS, C), lambda i: (i, 0)),
            pl.BlockSpec((ROWS, 1), lambda i: (i, 0)),
        ],
        out_specs=pl.BlockSpec((ROWS, C), lambda i: (i, 0)),
        out_shape=jax.ShapeDtypeStruct((B * F, C), input_features.dtype),
    )(x2, keep_col)
    return out.reshape(B, F, C)


# speedup vs baseline: 1.8995x; 1.8995x over previous
import functools

import jax
import jax.numpy as jnp
from jax import lax
from jax.experimental import pallas as pl

AUG_P = 0.5
DROP_P = 0.15
MIN_FEATURES = 1


def _mask_kernel(s_ref, m_ref, aug_ref, keep_ref, *, F, B):
    m = m_ref[...].T > 0
    bits = lax.bitcast_convert_type(s_ref[...].T, jnp.int32)
    bits = jnp.where(m, bits, jnp.int32(0x7FFFFFFF))

    n_avail = jnp.sum(m.astype(jnp.int32), axis=0, keepdims=True)
    k = (n_avail.astype(jnp.float32) * DROP_P).astype(jnp.int32)
    k = jnp.minimum(k, n_avail - MIN_FEATURES)
    aug = aug_ref[...].T < AUG_P
    k = jnp.where((n_avail > MIN_FEATURES) & aug & (k > 0), k, 0)

    ans = jnp.zeros((1, B), jnp.int32)
    for bit in range(30, -1, -1):
        test = ans + jnp.int32(1 << bit)
        cnt = jnp.sum((bits < test).astype(jnp.int32), axis=0, keepdims=True)
        ans = jnp.where(cnt < k, test, ans)

    c_lt = jnp.sum((bits < ans).astype(jnp.int32), axis=0, keepdims=True)
    eq = bits == ans
    fi = lax.broadcasted_iota(jnp.int32, (F, F), 0)
    fj = lax.broadcasted_iota(jnp.int32, (F, F), 1)
    tril = (fj < fi).astype(jnp.float32)
    eq_before = jax.lax.dot(
        tril, eq.astype(jnp.float32), precision=jax.lax.Precision.HIGHEST
    ).astype(jnp.int32)
    drop = m & ((bits < ans) | (eq & ((c_lt + eq_before) < k)))
    keep_ref[...] = (1.0 - drop.astype(jnp.float32)).T


def _copy_kernel(x_ref, k_ref, o_ref):
    o_ref[...] = x_ref[...] * k_ref[...][:, :, None]


def kernel(input_features, attention_mask):
    B, F, C = input_features.shape
    key = jax.random.key(42)
    k1, k2 = jax.random.split(key)
    aug_u = jax.random.uniform(k1, (B,)).reshape(B, 1)
    scores = jax.random.uniform(k2, (B, F))
    mask_i32 = attention_mask.astype(jnp.int32)

    keep = pl.pallas_call(
        functools.partial(_mask_kernel, F=F, B=B),
        out_shape=jax.ShapeDtypeStruct((B, F), jnp.float32),
    )(scores, mask_i32, aug_u)

    BBLK = 32
    grid = (B // BBLK,)
    out = pl.pallas_call(
        _copy_kernel,
        grid=grid,
        in_specs=[
            pl.BlockSpec((BBLK, F, C), lambda i: (i, 0, 0)),
            pl.BlockSpec((BBLK, F), lambda i: (i, 0)),
        ],
        out_specs=pl.BlockSpec((BBLK, F, C), lambda i: (i, 0, 0)),
        out_shape=jax.ShapeDtypeStruct((B, F, C), input_features.dtype),
    )(input_features, keep)
    return out
